# baseline (speedup 1.0000x reference)
import functools

import jax
import jax.numpy as jnp
from jax.experimental import pallas as pl

_TOKENS = 8192
_BLK = 512


def _split(a):
    hi = a.astype(jnp.bfloat16)
    lo = (a - hi.astype(jnp.float32)).astype(jnp.bfloat16)
    return hi, lo


def _dot3(xh, xl, wh_ref, wl_ref):
    f32 = jnp.float32
    main = jnp.dot(xh, wh_ref[...], preferred_element_type=f32)
    corr = (jnp.dot(xh, wl_ref[...], preferred_element_type=f32)
            + jnp.dot(xl, wh_ref[...], preferred_element_type=f32))
    return main + corr


def _router_body(x_ref, u_ref, w1h_ref, w1l_ref, b1_ref, w2h_ref, w2l_ref,
                 b2_ref, tab_ref, shp_ref, idx_ref):
    x = x_ref[...]
    xh, xl = _split(x)
    h = jnp.maximum(_dot3(xh, xl, w1h_ref, w1l_ref) + b1_ref[...], 0.0)
    hh, hl = _split(h)
    s = _dot3(hh, hl, w2h_ref, w2l_ref) + b2_ref[...]
    logits = jax.nn.sigmoid(s)
    u = jnp.clip(u_ref[...], 1e-10, 1.0 - 1e-10)
    g = -jnp.log(-jnp.log(u))
    score = logits + g
    m = jnp.max(score, axis=-1, keepdims=True)
    iota = jax.lax.broadcasted_iota(jnp.int32, score.shape, 1)
    idx = jnp.min(jnp.where(score == m, iota, score.shape[-1]),
                  axis=-1, keepdims=True)
    one_hot = (iota == idx).astype(jnp.bfloat16)
    shp_ref[...] = jnp.dot(one_hot, tab_ref[...],
                           preferred_element_type=jnp.float32)
    idx_ref[...] = idx


@functools.partial(jax.jit, static_argnames=())
def kernel(state, gumbel_u, W1, b1, W2, b2, shape_table):
    tokens, obs_dim = state.shape
    hidden = W1.shape[1]
    n_arcs = W2.shape[1]
    tab_w = shape_table.shape[1]
    w1h, w1l = _split(W1)
    w2h, w2l = _split(W2)
    grid = (tokens // _BLK,)
    shp, idx2 = pl.pallas_call(
        _router_body,
        grid=grid,
        in_specs=[
            pl.BlockSpec((_BLK, obs_dim), lambda i: (i, 0)),
            pl.BlockSpec((_BLK, n_arcs), lambda i: (i, 0)),
            pl.BlockSpec((obs_dim, hidden), lambda i: (0, 0)),
            pl.BlockSpec((obs_dim, hidden), lambda i: (0, 0)),
            pl.BlockSpec((1, hidden), lambda i: (0, 0)),
            pl.BlockSpec((hidden, n_arcs), lambda i: (0, 0)),
            pl.BlockSpec((hidden, n_arcs), lambda i: (0, 0)),
            pl.BlockSpec((1, n_arcs), lambda i: (0, 0)),
            pl.BlockSpec((n_arcs, tab_w), lambda i: (0, 0)),
        ],
        out_specs=[
            pl.BlockSpec((_BLK, tab_w), lambda i: (i, 0)),
            pl.BlockSpec((_BLK, 1), lambda i: (i, 0)),
        ],
        out_shape=[
            jax.ShapeDtypeStruct((tokens, tab_w), jnp.float32),
            jax.ShapeDtypeStruct((tokens, 1), jnp.int32),
        ],
    )(state, gumbel_u, w1h, w1l, b1.reshape(1, -1), w2h, w2l,
      b2.reshape(1, -1), shape_table.astype(jnp.bfloat16))
    return shp, idx2.reshape(tokens)


# speedup vs baseline: 1.5246x; 1.5246x over previous
import functools

import jax
import jax.numpy as jnp
from jax.experimental import pallas as pl

_BLK = 512
_LANES = 128


def _router_body(x_ref, u_ref, w1_ref, b1_ref, w2_ref, b2_ref,
                 tab_ref, out_ref):
    n_arcs = u_ref.shape[-1]
    x = x_ref[...]
    h = jnp.maximum(
        jnp.dot(x, w1_ref[...], preferred_element_type=jnp.float32)
        + b1_ref[...], 0.0)
    s = (jnp.dot(h, w2_ref[...], preferred_element_type=jnp.float32)
         + b2_ref[...])
    logits = jax.nn.sigmoid(s[:, :n_arcs])
    u = jnp.clip(u_ref[...], 1e-10, 1.0 - 1e-10)
    g = -jnp.log(-jnp.log(u))
    score = logits + g
    m = jnp.max(score, axis=-1, keepdims=True)
    iota = jax.lax.broadcasted_iota(jnp.int32, score.shape, 1)
    idx = jnp.min(jnp.where(score == m, iota, n_arcs),
                  axis=-1, keepdims=True)
    iota_w = jax.lax.broadcasted_iota(jnp.int32, s.shape, 1)
    one_hot = (iota_w == idx).astype(jnp.bfloat16)
    gathered = jnp.dot(one_hot, tab_ref[...],
                       preferred_element_type=jnp.float32)
    iota_out = jax.lax.broadcasted_iota(jnp.int32, gathered.shape, 1)
    is_idx_col = (iota_out == 11)
    out_ref[...] = jnp.where(is_idx_col, idx.astype(jnp.float32), gathered)


@functools.partial(jax.jit, static_argnames=())
def kernel(state, gumbel_u, W1, b1, W2, b2, shape_table):
    tokens, obs_dim = state.shape
    hidden = W1.shape[1]
    n_arcs = W2.shape[1]
    tab_w = shape_table.shape[1]
    hid_p = (hidden + _LANES - 1) // _LANES * _LANES
    arc_p = (n_arcs + _LANES - 1) // _LANES * _LANES
    w1p = jnp.pad(W1, ((0, 0), (0, hid_p - hidden)))
    b1p = jnp.pad(b1, (0, hid_p - hidden)).reshape(1, hid_p)
    w2p = jnp.pad(W2, ((0, hid_p - hidden), (0, arc_p - n_arcs)))
    b2p = jnp.pad(b2, (0, arc_p - n_arcs)).reshape(1, arc_p)
    tabp = jnp.pad(shape_table.astype(jnp.bfloat16),
                   ((0, arc_p - n_arcs), (0, _LANES - tab_w)))
    grid = (tokens // _BLK,)
    out = pl.pallas_call(
        _router_body,
        grid=grid,
        in_specs=[
            pl.BlockSpec((_BLK, obs_dim), lambda i: (i, 0)),
            pl.BlockSpec((_BLK, n_arcs), lambda i: (i, 0)),
            pl.BlockSpec((obs_dim, hid_p), lambda i: (0, 0)),
            pl.BlockSpec((1, hid_p), lambda i: (0, 0)),
            pl.BlockSpec((hid_p, arc_p), lambda i: (0, 0)),
            pl.BlockSpec((1, arc_p), lambda i: (0, 0)),
            pl.BlockSpec((arc_p, _LANES), lambda i: (0, 0)),
        ],
        out_specs=pl.BlockSpec((_BLK, _LANES), lambda i: (i, 0)),
        out_shape=jax.ShapeDtypeStruct((tokens, _LANES), jnp.float32),
    )(state, gumbel_u, w1p, b1p, w2p, b2p, tabp)
    return out[:, :tab_w], out[:, tab_w].astype(jnp.int32)


# speedup vs baseline: 2.6812x; 1.7586x over previous
import functools

import jax
import jax.numpy as jnp
from jax.experimental import pallas as pl

_BLK = 512
_OUT_ROWS = 16


def _router_body(x_ref, ut_ref, w1t_ref, b1_ref, w2t_ref, b2_ref,
                 tabt_ref, out_ref):
    n_arcs = ut_ref.shape[0]
    arc_p = tabt_ref.shape[1]
    blk = x_ref.shape[0]
    f32 = jnp.float32
    ht = jnp.maximum(
        jax.lax.dot_general(w1t_ref[...], x_ref[...],
                            (((1,), (1,)), ((), ())),
                            preferred_element_type=f32) + b1_ref[...], 0.0)
    st = (jax.lax.dot_general(w2t_ref[...], ht,
                              (((1,), (0,)), ((), ())),
                              preferred_element_type=f32) + b2_ref[...])
    logits = jax.nn.sigmoid(st)
    u = jnp.clip(ut_ref[...], 1e-10, 1.0 - 1e-10)
    g = -jnp.log(-jnp.log(u))
    score = logits + g
    m = jnp.max(score, axis=0, keepdims=True)
    iota = jax.lax.broadcasted_iota(jnp.int32, score.shape, 0)
    idx = jnp.min(jnp.where(score == m, iota, n_arcs),
                  axis=0, keepdims=True)
    iota_p = jax.lax.broadcasted_iota(jnp.int32, (arc_p, blk), 0)
    one_hot = (iota_p == idx).astype(jnp.bfloat16)
    gathered = jax.lax.dot_general(tabt_ref[...], one_hot,
                                   (((1,), (0,)), ((), ())),
                                   preferred_element_type=f32)
    iota_out = jax.lax.broadcasted_iota(jnp.int32, gathered.shape, 0)
    out_ref[...] = jnp.where(iota_out == 11, idx.astype(f32), gathered)


@functools.partial(jax.jit, static_argnames=())
def kernel(state, gumbel_u, W1, b1, W2, b2, shape_table):
    tokens, obs_dim = state.shape
    hidden = W1.shape[1]
    n_arcs = W2.shape[1]
    tab_w = shape_table.shape[1]
    arc_p = (n_arcs + 127) // 128 * 128
    ut = gumbel_u.T
    w1t = W1.T
    w2t = W2.T
    tabt = jnp.pad(shape_table.T.astype(jnp.bfloat16),
                   ((0, _OUT_ROWS - tab_w), (0, arc_p - n_arcs)))
    b1c = b1.reshape(hidden, 1)
    b2c = b2.reshape(n_arcs, 1)
    grid = (tokens // _BLK,)
    out = pl.pallas_call(
        _router_body,
        grid=grid,
        in_specs=[
            pl.BlockSpec((_BLK, obs_dim), lambda i: (i, 0)),
            pl.BlockSpec((n_arcs, _BLK), lambda i: (0, i)),
            pl.BlockSpec((hidden, obs_dim), lambda i: (0, 0)),
            pl.BlockSpec((hidden, 1), lambda i: (0, 0)),
            pl.BlockSpec((n_arcs, hidden), lambda i: (0, 0)),
            pl.BlockSpec((n_arcs, 1), lambda i: (0, 0)),
            pl.BlockSpec((_OUT_ROWS, arc_p), lambda i: (0, 0)),
        ],
        out_specs=pl.BlockSpec((_OUT_ROWS, _BLK), lambda i: (0, i)),
        out_shape=jax.ShapeDtypeStruct((_OUT_ROWS, tokens), jnp.float32),
    )(state, ut, w1t, b1c, w2t, b2c, tabt)
    return out[:tab_w, :].T, out[tab_w, :].astype(jnp.int32)


# speedup vs baseline: 2.8352x; 1.0575x over previous
import functools

import jax
import jax.numpy as jnp
from jax.experimental import pallas as pl

_BLK = 512
_OUT_ROWS = 16


def _router_body(x_ref, ut_ref, w1t_ref, b1_ref, w2t_ref, b2_ref,
                 tabt_ref, shp_ref, idx_ref):
    n_arcs = ut_ref.shape[0]
    arc_p = tabt_ref.shape[1]
    blk = x_ref.shape[0]
    f32 = jnp.float32
    ht = jnp.maximum(
        jax.lax.dot_general(w1t_ref[...], x_ref[...],
                            (((1,), (1,)), ((), ())),
                            preferred_element_type=f32) + b1_ref[...], 0.0)
    st = (jax.lax.dot_general(w2t_ref[...], ht,
                              (((1,), (0,)), ((), ())),
                              preferred_element_type=f32) + b2_ref[...])
    logits = jax.nn.sigmoid(st)
    u = jnp.clip(ut_ref[...], 1e-10, 1.0 - 1e-10)
    g = -jnp.log(-jnp.log(u))
    score = logits + g
    m = jnp.max(score, axis=0, keepdims=True)
    iota = jax.lax.broadcasted_iota(jnp.int32, score.shape, 0)
    idx = jnp.min(jnp.where(score == m, iota, n_arcs),
                  axis=0, keepdims=True)
    iota_p = jax.lax.broadcasted_iota(jnp.int32, (arc_p, blk), 0)
    one_hot = (iota_p == idx).astype(jnp.bfloat16)
    shp_ref[...] = jax.lax.dot_general(tabt_ref[...], one_hot,
                                       (((1,), (0,)), ((), ())),
                                       preferred_element_type=f32)
    idx_ref[...] = idx


@functools.partial(jax.jit, static_argnames=())
def kernel(state, gumbel_u, W1, b1, W2, b2, shape_table):
    tokens, obs_dim = state.shape
    hidden = W1.shape[1]
    n_arcs = W2.shape[1]
    tab_w = shape_table.shape[1]
    arc_p = (n_arcs + 127) // 128 * 128
    ut = gumbel_u.T
    w1t = W1.T
    w2t = W2.T
    tabt = jnp.pad(shape_table.T.astype(jnp.bfloat16),
                   ((0, 0), (0, arc_p - n_arcs)))
    b1c = b1.reshape(hidden, 1)
    b2c = b2.reshape(n_arcs, 1)
    grid = (tokens // _BLK,)
    out = pl.pallas_call(
        _router_body,
        grid=grid,
        in_specs=[
            pl.BlockSpec((_BLK, obs_dim), lambda i: (i, 0)),
            pl.BlockSpec((n_arcs, _BLK), lambda i: (0, i)),
            pl.BlockSpec((hidden, obs_dim), lambda i: (0, 0)),
            pl.BlockSpec((hidden, 1), lambda i: (0, 0)),
            pl.BlockSpec((n_arcs, hidden), lambda i: (0, 0)),
            pl.BlockSpec((n_arcs, 1), lambda i: (0, 0)),
            pl.BlockSpec((tab_w, arc_p), lambda i: (0, 0)),
        ],
        out_specs=[
            pl.BlockSpec((tab_w, _BLK), lambda i: (0, i)),
            pl.BlockSpec((1, _BLK), lambda i: (0, i)),
        ],
        out_shape=[
            jax.ShapeDtypeStruct((tab_w, tokens), jnp.float32),
            jax.ShapeDtypeStruct((1, tokens), jnp.int32),
        ],
    )(state, ut, w1t, b1c, w2t, b2c, tabt)
    shp, idx = out
    return shp.T, idx.reshape(tokens)


# speedup vs baseline: 3.1340x; 1.1054x over previous
import functools

import jax
import jax.numpy as jnp
from jax.experimental import pallas as pl

_BLK = 1024
_OUT_ROWS = 16


def _router_body(x_ref, ut_ref, w1t_ref, b1_ref, w2t_ref, b2_ref,
                 tabt_ref, shp_ref, idx_ref):
    n_arcs = ut_ref.shape[0]
    arc_p = tabt_ref.shape[1]
    blk = x_ref.shape[0]
    f32 = jnp.float32
    ht = jnp.maximum(
        jax.lax.dot_general(w1t_ref[...], x_ref[...],
                            (((1,), (1,)), ((), ())),
                            preferred_element_type=f32) + b1_ref[...], 0.0)
    st = (jax.lax.dot_general(w2t_ref[...], ht,
                              (((1,), (0,)), ((), ())),
                              preferred_element_type=f32) + b2_ref[...])
    logits = jax.nn.sigmoid(st)
    u = jnp.clip(ut_ref[...], 1e-10, 1.0 - 1e-10)
    g = -jnp.log(-jnp.log(u))
    score = logits + g
    m = jnp.max(score, axis=0, keepdims=True)
    iota = jax.lax.broadcasted_iota(jnp.int32, score.shape, 0)
    idx = jnp.min(jnp.where(score == m, iota, n_arcs),
                  axis=0, keepdims=True)
    iota_p = jax.lax.broadcasted_iota(jnp.int32, (arc_p, blk), 0)
    one_hot = (iota_p == idx).astype(jnp.bfloat16)
    shp_ref[...] = jax.lax.dot_general(tabt_ref[...], one_hot,
                                       (((1,), (0,)), ((), ())),
                                       preferred_element_type=f32)
    idx_ref[...] = idx


@functools.partial(jax.jit, static_argnames=())
def kernel(state, gumbel_u, W1, b1, W2, b2, shape_table):
    tokens, obs_dim = state.shape
    hidden = W1.shape[1]
    n_arcs = W2.shape[1]
    tab_w = shape_table.shape[1]
    arc_p = (n_arcs + 127) // 128 * 128
    ut = gumbel_u.T
    w1t = W1.T
    w2t = W2.T
    tabt = jnp.pad(shape_table.T.astype(jnp.bfloat16),
                   ((0, 0), (0, arc_p - n_arcs)))
    b1c = b1.reshape(hidden, 1)
    b2c = b2.reshape(n_arcs, 1)
    grid = (tokens // _BLK,)
    out = pl.pallas_call(
        _router_body,
        grid=grid,
        in_specs=[
            pl.BlockSpec((_BLK, obs_dim), lambda i: (i, 0)),
            pl.BlockSpec((n_arcs, _BLK), lambda i: (0, i)),
            pl.BlockSpec((hidden, obs_dim), lambda i: (0, 0)),
            pl.BlockSpec((hidden, 1), lambda i: (0, 0)),
            pl.BlockSpec((n_arcs, hidden), lambda i: (0, 0)),
            pl.BlockSpec((n_arcs, 1), lambda i: (0, 0)),
            pl.BlockSpec((tab_w, arc_p), lambda i: (0, 0)),
        ],
        out_specs=[
            pl.BlockSpec((tab_w, _BLK), lambda i: (0, i)),
            pl.BlockSpec((1, _BLK), lambda i: (0, i)),
        ],
        out_shape=[
            jax.ShapeDtypeStruct((tab_w, tokens), jnp.float32),
            jax.ShapeDtypeStruct((1, tokens), jnp.int32),
        ],
    )(state, ut, w1t, b1c, w2t, b2c, tabt)
    shp, idx = out
    return shp.T, idx.reshape(tokens)
